# Initial kernel scaffold; baseline (speedup 1.0000x reference)
import jax
import jax.numpy as jnp
from jax.experimental import pallas as pl


def kernel(x, edge_index, W1l, b1, W1r, W2l, b2, W2r, Wfc, bfc):
    raise NotImplementedError("write your pallas kernel here")


# speedup vs baseline: 5.2048x; 5.2048x over previous
import functools

import jax
import jax.numpy as jnp
from jax import lax
from jax.experimental import pallas as pl
from jax.experimental.pallas import tpu as pltpu
from jax.experimental.pallas import tpu_sc as plsc

N = 10000
E = 320000
NT = 10240
R = 640
GRID = NT // R
NC = 2
NS = 16
NW = NC * NS
EW = E // NW
K = 80
NB = EW // K
ROWS_PER_TILE = NT // NS


def _dense1_body(x_ref, wl_ref, wr_ref, b1_ref, xl_ref, xr_ref):
    x = x_ref[...]
    xl_ref[...] = jnp.dot(x, wl_ref[...], preferred_element_type=jnp.float32)
    xr_ref[...] = (jnp.dot(x, wr_ref[...], preferred_element_type=jnp.float32)
                   + b1_ref[...])


def _dense1(x_pad, W1l_T, W1r_T, b1):
    return pl.pallas_call(
        _dense1_body,
        grid=(GRID,),
        in_specs=[
            pl.BlockSpec((R, 128), lambda i: (i, 0)),
            pl.BlockSpec((128, 128), lambda i: (0, 0)),
            pl.BlockSpec((128, 128), lambda i: (0, 0)),
            pl.BlockSpec((1, 128), lambda i: (0, 0)),
        ],
        out_specs=[
            pl.BlockSpec((R, 128), lambda i: (i, 0)),
            pl.BlockSpec((R, 128), lambda i: (i, 0)),
        ],
        out_shape=[
            jax.ShapeDtypeStruct((NT, 128), jnp.float32),
            jax.ShapeDtypeStruct((NT, 128), jnp.float32),
        ],
    )(x_pad, W1l_T, W1r_T, b1)


def _dense2_body(agg_ref, cnt_ref, xr_ref, w2l_ref, w2r_ref, b2_ref,
                 hl_ref, hr_ref):
    agg = agg_ref[0] + agg_ref[1]
    cnt = cnt_ref[0] + cnt_ref[1]
    inv = 1.0 / jnp.maximum(cnt, 1.0)
    h1 = jnp.maximum(agg * inv[:, None] + xr_ref[...], 0.0)
    hl_ref[...] = jnp.dot(h1, w2l_ref[...], preferred_element_type=jnp.float32)
    hr_ref[...] = (jnp.dot(h1, w2r_ref[...], preferred_element_type=jnp.float32)
                   + b2_ref[...])


def _dense2(aggP, cntP, xr, W2l_T, W2r_T, b2):
    return pl.pallas_call(
        _dense2_body,
        grid=(GRID,),
        in_specs=[
            pl.BlockSpec((2, R, 128), lambda i: (0, i, 0)),
            pl.BlockSpec((2, R), lambda i: (0, i)),
            pl.BlockSpec((R, 128), lambda i: (i, 0)),
            pl.BlockSpec((128, 64), lambda i: (0, 0)),
            pl.BlockSpec((128, 64), lambda i: (0, 0)),
            pl.BlockSpec((1, 64), lambda i: (0, 0)),
        ],
        out_specs=[
            pl.BlockSpec((R, 64), lambda i: (i, 0)),
            pl.BlockSpec((R, 64), lambda i: (i, 0)),
        ],
        out_shape=[
            jax.ShapeDtypeStruct((NT, 64), jnp.float32),
            jax.ShapeDtypeStruct((NT, 64), jnp.float32),
        ],
    )(aggP, cntP, xr, W2l_T, W2r_T, b2)


def _dense3_body(agg_ref, cnt_ref, hr_ref, wfc_ref, bfc_ref, out_ref):
    agg = agg_ref[0] + agg_ref[1]
    cnt = cnt_ref[0] + cnt_ref[1]
    inv = 1.0 / jnp.maximum(cnt, 1.0)
    h2 = jnp.maximum(agg * inv[:, None] + hr_ref[...], 0.0)
    logit = jnp.sum(h2 * wfc_ref[...], axis=1, keepdims=True) + bfc_ref[...]
    out_ref[...] = jax.nn.sigmoid(logit)


def _dense3(agg2P, cntP, hr, Wfc, bfc):
    return pl.pallas_call(
        _dense3_body,
        grid=(GRID,),
        in_specs=[
            pl.BlockSpec((2, R, 64), lambda i: (0, i, 0)),
            pl.BlockSpec((2, R), lambda i: (0, i)),
            pl.BlockSpec((R, 64), lambda i: (i, 0)),
            pl.BlockSpec((1, 64), lambda i: (0, 0)),
            pl.BlockSpec((1, 1), lambda i: (0, 0)),
        ],
        out_specs=pl.BlockSpec((R, 1), lambda i: (i, 0)),
        out_shape=jax.ShapeDtypeStruct((NT, 1), jnp.float32),
    )(agg2P, cntP, hr, Wfc, bfc)


def _make_agg(D, with_cnt):
    mesh = plsc.VectorSubcoreMesh(
        core_axis_name="c", subcore_axis_name="s",
        num_cores=NC, num_subcores=NS)

    out_type = [jax.ShapeDtypeStruct((NC, NT, D), jnp.float32)]
    if with_cnt:
        out_type.append(jax.ShapeDtypeStruct((NC, NT), jnp.float32))

    scratch = [
        pltpu.VMEM((K,), jnp.int32),
        pltpu.VMEM((K,), jnp.int32),
        pltpu.VMEM((K, D), jnp.float32),
        pltpu.VMEM((K,), jnp.float32),
        pltpu.VMEM_SHARED((NT, D), jnp.float32),
        pltpu.VMEM_SHARED((NT,), jnp.float32),
        pltpu.SemaphoreType.DMA,
    ]

    @functools.partial(pl.kernel, out_type=out_type, mesh=mesh,
                       scratch_types=scratch,
                       compiler_params=pltpu.CompilerParams(
                           use_tc_tiling_on_sc=False))
    def agg_kernel(table, src, dst, zrows, zcnt, *refs):
        if with_cnt:
            out, cnt_out = refs[0], refs[1]
            rest = refs[2:]
        else:
            out = refs[0]
            rest = refs[1:]
        srcv, dstv, rows, ones, acc, cacc, sem = rest

        cid = lax.axis_index("c")
        sid = lax.axis_index("s")
        wid = cid * NS + sid

        zbase = sid * ROWS_PER_TILE
        pltpu.sync_copy(zrows, acc.at[pl.ds(zbase, ROWS_PER_TILE)])
        if with_cnt:
            pltpu.sync_copy(zcnt, cacc.at[pl.ds(zbase, ROWS_PER_TILE)])
            for j in range(K // 16):
                ones[pl.ds(16 * j, 16)] = jnp.full((16,), 1.0, jnp.float32)
        plsc.subcore_barrier()

        base = wid * EW

        def body(i, carry):
            off = base + i * K
            pltpu.sync_copy(src.at[pl.ds(off, K)], srcv)
            pltpu.sync_copy(dst.at[pl.ds(off, K)], dstv)
            pltpu.async_copy(table.at[srcv], rows, sem).wait()
            pltpu.sync_copy(rows, acc.at[dstv], add=True)
            if with_cnt:
                pltpu.sync_copy(ones, cacc.at[dstv], add=True)
            return carry

        lax.fori_loop(0, NB, body, 0)
        plsc.subcore_barrier()

        pltpu.sync_copy(acc.at[pl.ds(zbase, ROWS_PER_TILE)],
                        out.at[cid, pl.ds(zbase, ROWS_PER_TILE)])
        if with_cnt:
            pltpu.sync_copy(cacc.at[pl.ds(zbase, ROWS_PER_TILE)],
                            cnt_out.at[cid, pl.ds(zbase, ROWS_PER_TILE)])

    return agg_kernel


_agg128 = _make_agg(128, with_cnt=True)
_agg64 = _make_agg(64, with_cnt=False)


def kernel(x, edge_index, W1l, b1, W1r, W2l, b2, W2r, Wfc, bfc):
    x_pad = jnp.pad(x, ((0, NT - N), (0, 0)))
    src = edge_index[0]
    dst = edge_index[1]
    zrows128 = jnp.zeros((ROWS_PER_TILE, 128), jnp.float32)
    zrows64 = jnp.zeros((ROWS_PER_TILE, 64), jnp.float32)
    zcnt = jnp.zeros((ROWS_PER_TILE,), jnp.float32)

    xl, xr = _dense1(x_pad, W1l.T, W1r.T, b1.reshape(1, 128))
    aggP, cntP = _agg128(xl, src, dst, zrows128, zcnt)
    hl, hr = _dense2(aggP, cntP, xr, W2l.T, W2r.T, b2.reshape(1, 64))
    (agg2P,) = _agg64(hl, src, dst, zrows64, zcnt)
    out = _dense3(agg2P, cntP, hr, Wfc.reshape(1, 64), bfc.reshape(1, 1))
    return out[:N, 0]
